# baseline (speedup 1.0000x reference)
import functools

import numpy as np
import jax
import jax.numpy as jnp
from jax import lax
from jax.experimental import pallas as pl
from jax.experimental.pallas import tpu as pltpu
from jax.experimental.pallas import tpu_sc as plsc

_B, _C, _H, _W = 8, 96, 32, 32
_N = _H * _W
_K = 9
_BN_ROWS = _B * _N
_EPS = 1e-5
_NW = 32
_RPW = _BN_ROWS // _NW
_NBUF = 4


def _sincos_1d_np(embed_dim, pos):
    omega = np.arange(embed_dim // 2, dtype=np.float64) / (embed_dim / 2.0)
    omega = 1.0 / (10000.0 ** omega)
    out = pos.reshape(-1)[:, None] * omega[None, :]
    return np.concatenate([np.sin(out), np.cos(out)], axis=1)


def _rel_pos_const(embed_dim, gh, gw):
    gx = np.arange(gw, dtype=np.float64)
    gy = np.arange(gh, dtype=np.float64)
    gX, gY = np.meshgrid(gx, gy)
    emb_w = _sincos_1d_np(embed_dim // 2, gX)
    emb_h = _sincos_1d_np(embed_dim // 2, gY)
    pos = np.concatenate([emb_h, emb_w], axis=1)
    rel = 2.0 * (pos @ pos.T) / float(embed_dim)
    return (-rel).astype(np.float32)


_REL_POS_NP = _rel_pos_const(_C, _H, _W)


def _bn_cols(v, gamma, beta):
    mu = jnp.mean(v, axis=0, keepdims=True)
    var = jnp.mean((v - mu) ** 2, axis=0, keepdims=True)
    return (v - mu) / jnp.sqrt(var + _EPS) * gamma + beta


def _dot_bf16(a, b, dims):
    return lax.dot_general(a.astype(jnp.bfloat16), b.astype(jnp.bfloat16),
                           dims, preferred_element_type=jnp.float32)


def _stage_a(xt_ref, w1_ref, b1_ref, g1_ref, bb1_ref, y_ref, yn_ref):
    xt = xt_ref[...]
    y_pre = _dot_bf16(xt, w1_ref[...], (((1,), (1,)), ((), ())))
    y_pre = y_pre + b1_ref[...]
    y = _bn_cols(y_pre, g1_ref[...], bb1_ref[...])
    nrm = jnp.sqrt(jnp.sum(y * y, axis=1, keepdims=True))
    yn = y / jnp.maximum(nrm, 1e-12)
    y_ref[...] = jnp.concatenate([y, jnp.zeros((_BN_ROWS, 128 - _C), jnp.float32)], axis=1)
    yn_ref[...] = yn


def _stage_b(yn_ref, rp_ref, idx_ref):
    yn = yn_ref[...]
    yn2 = yn * yn
    sq_col = jnp.sum(yn2, axis=1, keepdims=True)
    ones_row = jnp.ones((1, _C), jnp.float32)
    s_hi = yn2.astype(jnp.bfloat16)
    r1 = yn2 - s_hi.astype(jnp.float32)
    s_mid = r1.astype(jnp.bfloat16)
    s_lo = r1 - s_mid.astype(jnp.float32)
    dims_t = (((1,), (1,)), ((), ()))
    sq_row = (_dot_bf16(ones_row, s_hi, dims_t)
              + _dot_bf16(ones_row, s_mid, dims_t)
              + _dot_bf16(ones_row, s_lo, dims_t))
    g = _dot_bf16(yn, yn, dims_t)
    d = (sq_col - 2.0 * g) + (sq_row + rp_ref[...])
    iota = lax.broadcasted_iota(jnp.int32, (_N, _N), 1)
    lane16 = lax.broadcasted_iota(jnp.int32, (_N, 16), 1)
    base = pl.program_id(0) * _N
    acc_idx = jnp.zeros((_N, 16), jnp.int32)
    for k in range(_K):
        m = jnp.min(d, axis=1, keepdims=True)
        idx = jnp.min(jnp.where(d == m, iota, _N), axis=1, keepdims=True)
        if k == 0:
            acc_idx = jnp.broadcast_to(idx, (_N, 16))
        else:
            acc_idx = jnp.where(lane16 == k, idx, acc_idx)
        d = jnp.where(iota == idx, jnp.inf, d)
    idx_ref[...] = acc_idx + base


def _sc_gather_max(y_hbm, idx_hbm, out_hbm, idxv, yv, outv,
                   gb0, gb1, gb2, gb3, s0, s1, s2, s3):
    c = lax.axis_index("c")
    s = lax.axis_index("s")
    wid = s * 2 + c
    base = wid * _RPW
    pltpu.sync_copy(idx_hbm.at[pl.ds(base, _RPW)], idxv)
    pltpu.sync_copy(y_hbm.at[pl.ds(base, _RPW)], yv)
    gbs = (gb0, gb1, gb2, gb3)
    sems = (s0, s1, s2, s3)
    for b in range(_NBUF):
        pltpu.async_copy(y_hbm.at[idxv.at[b]], gbs[b], sems[b])

    def chunk(i, carry):
        g = i * _NBUF
        for b in range(_NBUF):
            r = g + b
            pltpu.make_async_copy(y_hbm.at[idxv.at[0]], gbs[b], sems[b]).wait()
            gb = gbs[b]
            for cv in range(_C // 16):
                acc = gb[0, pl.ds(cv * 16, 16)]
                for k in range(1, 16):
                    acc = jnp.maximum(acc, gb[k, pl.ds(cv * 16, 16)])
                outv[r, pl.ds(cv * 16, 16)] = acc - yv[r, pl.ds(cv * 16, 16)]

            @pl.when(r + _NBUF < _RPW)
            def _():
                pltpu.async_copy(y_hbm.at[idxv.at[r + _NBUF]], gbs[b], sems[b])
        return carry

    lax.fori_loop(0, _RPW // _NBUF, chunk, 0)
    pltpu.sync_copy(outv, out_hbm.at[pl.ds(base, _RPW)])


def _stage_c(y_ref, rm_ref, xt_ref, wg_ref, bg_ref, gg_ref, bbg_ref,
             w2_ref, b2_ref, g2_ref, bb2_ref, out_ref):
    h = jnp.concatenate([y_ref[...][:, :_C], rm_ref[...]], axis=1)
    hg = _dot_bf16(h, wg_ref[...], (((1,), (0,)), ((), ()))) + bg_ref[...]
    hg = _bn_cols(hg, gg_ref[...], bbg_ref[...])
    hg = 0.5 * hg * (1.0 + lax.erf(hg * np.float32(1.0 / np.sqrt(2.0))))
    out = _dot_bf16(hg, w2_ref[...], (((1,), (1,)), ((), ()))) + b2_ref[...]
    out = _bn_cols(out, g2_ref[...], bb2_ref[...])
    out_ref[...] = out + xt_ref[...]


def kernel(x, W_fc1, b_fc1, g_bn1, b_bn1, W_g, b_g, g_bng, b_bng,
           W_fc2, b_fc2, g_bn2, b_bn2):
    xt = x.reshape(_B, _C, _N).transpose(0, 2, 1).reshape(_BN_ROWS, _C)
    rel_pos = jnp.asarray(_REL_POS_NP)

    y, yn = pl.pallas_call(
        _stage_a,
        out_shape=[jax.ShapeDtypeStruct((_BN_ROWS, 128), jnp.float32),
                   jax.ShapeDtypeStruct((_BN_ROWS, _C), jnp.float32)],
    )(xt, W_fc1, b_fc1.reshape(1, _C), g_bn1.reshape(1, _C),
      b_bn1.reshape(1, _C))

    nn_idx = pl.pallas_call(
        _stage_b,
        grid=(_B,),
        in_specs=[
            pl.BlockSpec((_N, _C), lambda i: (i, 0)),
            pl.BlockSpec((_N, _N), lambda i: (0, 0)),
        ],
        out_specs=pl.BlockSpec((_N, 16), lambda i: (i, 0)),
        out_shape=jax.ShapeDtypeStruct((_BN_ROWS, 16), jnp.int32),
        compiler_params=pltpu.CompilerParams(
            dimension_semantics=("arbitrary",)),
    )(yn, rel_pos)

    mesh = plsc.VectorSubcoreMesh(core_axis_name="c", subcore_axis_name="s")
    rel_max = pl.kernel(
        _sc_gather_max,
        mesh=mesh,
        out_type=jax.ShapeDtypeStruct((_BN_ROWS, _C), jnp.float32),
        scratch_types=[
            pltpu.VMEM((_RPW, 16), jnp.int32),
            pltpu.VMEM((_RPW, 128), jnp.float32),
            pltpu.VMEM((_RPW, _C), jnp.float32),
            pltpu.VMEM((16, 128), jnp.float32),
            pltpu.VMEM((16, 128), jnp.float32),
            pltpu.VMEM((16, 128), jnp.float32),
            pltpu.VMEM((16, 128), jnp.float32),
            pltpu.SemaphoreType.DMA,
            pltpu.SemaphoreType.DMA,
            pltpu.SemaphoreType.DMA,
            pltpu.SemaphoreType.DMA,
        ],
    )(y, nn_idx)

    wg_bd = jax.scipy.linalg.block_diag(*[W_g[g].T for g in range(4)])

    out = pl.pallas_call(
        _stage_c,
        out_shape=jax.ShapeDtypeStruct((_BN_ROWS, _C), jnp.float32),
    )(y, rel_max, xt, wg_bd, b_g.reshape(1, 2 * _C), g_bng.reshape(1, 2 * _C),
      b_bng.reshape(1, 2 * _C), W_fc2, b_fc2.reshape(1, _C),
      g_bn2.reshape(1, _C), b_bn2.reshape(1, _C))

    return out.reshape(_B, _N, _C).transpose(0, 2, 1).reshape(_B, _C, _H, _W)


# speedup vs baseline: 1.3548x; 1.3548x over previous
import functools

import numpy as np
import jax
import jax.numpy as jnp
from jax import lax
from jax.experimental import pallas as pl
from jax.experimental.pallas import tpu as pltpu
from jax.experimental.pallas import tpu_sc as plsc

_B, _C, _H, _W = 8, 96, 32, 32
_N = _H * _W
_K = 9
_BN_ROWS = _B * _N
_EPS = 1e-5
_NW = 32
_RPW = _BN_ROWS // _NW
_NBUF = 4


def _sincos_1d_np(embed_dim, pos):
    omega = np.arange(embed_dim // 2, dtype=np.float64) / (embed_dim / 2.0)
    omega = 1.0 / (10000.0 ** omega)
    out = pos.reshape(-1)[:, None] * omega[None, :]
    return np.concatenate([np.sin(out), np.cos(out)], axis=1)


def _rel_pos_const(embed_dim, gh, gw):
    gx = np.arange(gw, dtype=np.float64)
    gy = np.arange(gh, dtype=np.float64)
    gX, gY = np.meshgrid(gx, gy)
    emb_w = _sincos_1d_np(embed_dim // 2, gX)
    emb_h = _sincos_1d_np(embed_dim // 2, gY)
    pos = np.concatenate([emb_h, emb_w], axis=1)
    rel = 2.0 * (pos @ pos.T) / float(embed_dim)
    return (-rel).astype(np.float32)


_REL_POS_NP = _rel_pos_const(_C, _H, _W)


def _bn_cols(v, gamma, beta):
    mu = jnp.mean(v, axis=0, keepdims=True)
    var = jnp.mean((v - mu) ** 2, axis=0, keepdims=True)
    return (v - mu) / jnp.sqrt(var + _EPS) * gamma + beta


def _dot_bf16(a, b, dims):
    return lax.dot_general(a.astype(jnp.bfloat16), b.astype(jnp.bfloat16),
                           dims, preferred_element_type=jnp.float32)


def _stage_a(xt_ref, w1_ref, b1_ref, g1_ref, bb1_ref, y_ref, yn_ref):
    xt = xt_ref[...]
    y_pre = _dot_bf16(xt, w1_ref[...], (((1,), (1,)), ((), ())))
    y_pre = y_pre + b1_ref[...]
    y = _bn_cols(y_pre, g1_ref[...], bb1_ref[...])
    nrm = jnp.sqrt(jnp.sum(y * y, axis=1, keepdims=True))
    yn = y / jnp.maximum(nrm, 1e-12)
    y_ref[...] = jnp.concatenate([y, jnp.zeros((_BN_ROWS, 128 - _C), jnp.float32)], axis=1)
    yn_ref[...] = yn


def _stage_b(yn_ref, rp_ref, idx_ref):
    yn = yn_ref[...]
    yn2 = yn * yn
    sq_col = jnp.sum(yn2, axis=1, keepdims=True)
    ones_row = jnp.ones((1, _C), jnp.float32)
    s_hi = yn2.astype(jnp.bfloat16)
    r1 = yn2 - s_hi.astype(jnp.float32)
    s_mid = r1.astype(jnp.bfloat16)
    s_lo = r1 - s_mid.astype(jnp.float32)
    dims_t = (((1,), (1,)), ((), ()))
    sq_row = (_dot_bf16(ones_row, s_hi, dims_t)
              + _dot_bf16(ones_row, s_mid, dims_t)
              + _dot_bf16(ones_row, s_lo, dims_t))
    g = _dot_bf16(yn, yn, dims_t)
    d = (sq_col - 2.0 * g) + (sq_row + rp_ref[...])
    q = (jnp.clip(d, -8.0, 8.0) * 131072.0).astype(jnp.int32)
    iota = lax.broadcasted_iota(jnp.int32, (_N, _N), 1)
    p = (q << 10) | iota
    lane16 = lax.broadcasted_iota(jnp.int32, (_N, 16), 1)
    base = pl.program_id(0) * _N
    acc_idx = jnp.zeros((_N, 16), jnp.int32)
    for k in range(_K):
        m = jnp.min(p, axis=1, keepdims=True)
        idx = m & jnp.int32(1023)
        if k == 0:
            acc_idx = jnp.broadcast_to(idx, (_N, 16))
        else:
            acc_idx = jnp.where(lane16 == k, idx, acc_idx)
        p = jnp.where(p == m, jnp.int32(0x7FFFFFFF), p)
    idx_ref[...] = acc_idx + base


def _sc_gather_max(y_hbm, idx_hbm, out_hbm, idxv, outv,
                   gb0, gb1, gb2, gb3, s0, s1, s2, s3):
    c = lax.axis_index("c")
    s = lax.axis_index("s")
    wid = s * 2 + c
    base = wid * _RPW
    pltpu.sync_copy(idx_hbm.at[pl.ds(base, _RPW)], idxv)
    gbs = (gb0, gb1, gb2, gb3)
    sems = (s0, s1, s2, s3)
    for b in range(_NBUF):
        pltpu.async_copy(y_hbm.at[idxv.at[b, pl.ds(0, _K)]], gbs[b], sems[b])

    def chunk(i, carry):
        g = i * _NBUF
        for b in range(_NBUF):
            r = g + b
            pltpu.make_async_copy(
                y_hbm.at[idxv.at[0, pl.ds(0, _K)]], gbs[b], sems[b]).wait()
            gb = gbs[b]
            for cv in range(_C // 16):
                acc = gb[0, pl.ds(cv * 16, 16)]
                for k in range(1, _K):
                    acc = jnp.maximum(acc, gb[k, pl.ds(cv * 16, 16)])
                outv[r, pl.ds(cv * 16, 16)] = acc

            @pl.when(r + _NBUF < _RPW)
            def _():
                pltpu.async_copy(
                    y_hbm.at[idxv.at[r + _NBUF, pl.ds(0, _K)]], gbs[b], sems[b])
        return carry

    lax.fori_loop(0, _RPW // _NBUF, chunk, 0)
    pltpu.sync_copy(outv, out_hbm.at[pl.ds(base, _RPW)])


def _stage_c(y_ref, rm_ref, xt_ref, wg_ref, bg_ref, gg_ref, bbg_ref,
             w2_ref, b2_ref, g2_ref, bb2_ref, out_ref):
    y = y_ref[...][:, :_C]
    h = jnp.concatenate([y, rm_ref[...] - y], axis=1)
    hg = _dot_bf16(h, wg_ref[...], (((1,), (0,)), ((), ()))) + bg_ref[...]
    hg = _bn_cols(hg, gg_ref[...], bbg_ref[...])
    hg = 0.5 * hg * (1.0 + lax.erf(hg * np.float32(1.0 / np.sqrt(2.0))))
    out = _dot_bf16(hg, w2_ref[...], (((1,), (1,)), ((), ()))) + b2_ref[...]
    out = _bn_cols(out, g2_ref[...], bb2_ref[...])
    out_ref[...] = out + xt_ref[...]


def kernel(x, W_fc1, b_fc1, g_bn1, b_bn1, W_g, b_g, g_bng, b_bng,
           W_fc2, b_fc2, g_bn2, b_bn2):
    xt = x.reshape(_B, _C, _N).transpose(0, 2, 1).reshape(_BN_ROWS, _C)
    rel_pos = jnp.asarray(_REL_POS_NP)

    y, yn = pl.pallas_call(
        _stage_a,
        out_shape=[jax.ShapeDtypeStruct((_BN_ROWS, 128), jnp.float32),
                   jax.ShapeDtypeStruct((_BN_ROWS, _C), jnp.float32)],
    )(xt, W_fc1, b_fc1.reshape(1, _C), g_bn1.reshape(1, _C),
      b_bn1.reshape(1, _C))

    nn_idx = pl.pallas_call(
        _stage_b,
        grid=(_B,),
        in_specs=[
            pl.BlockSpec((_N, _C), lambda i: (i, 0)),
            pl.BlockSpec((_N, _N), lambda i: (0, 0)),
        ],
        out_specs=pl.BlockSpec((_N, 16), lambda i: (i, 0)),
        out_shape=jax.ShapeDtypeStruct((_BN_ROWS, 16), jnp.int32),
        compiler_params=pltpu.CompilerParams(
            dimension_semantics=("arbitrary",)),
    )(yn, rel_pos)

    mesh = plsc.VectorSubcoreMesh(core_axis_name="c", subcore_axis_name="s")
    rel_max = pl.kernel(
        _sc_gather_max,
        mesh=mesh,
        out_type=jax.ShapeDtypeStruct((_BN_ROWS, _C), jnp.float32),
        scratch_types=[
            pltpu.VMEM((_RPW, 16), jnp.int32),
            pltpu.VMEM((_RPW, _C), jnp.float32),
            pltpu.VMEM((_K, 128), jnp.float32),
            pltpu.VMEM((_K, 128), jnp.float32),
            pltpu.VMEM((_K, 128), jnp.float32),
            pltpu.VMEM((_K, 128), jnp.float32),
            pltpu.SemaphoreType.DMA,
            pltpu.SemaphoreType.DMA,
            pltpu.SemaphoreType.DMA,
            pltpu.SemaphoreType.DMA,
        ],
    )(y, nn_idx)

    wg_bd = jax.scipy.linalg.block_diag(*[W_g[g].T for g in range(4)])

    out = pl.pallas_call(
        _stage_c,
        out_shape=jax.ShapeDtypeStruct((_BN_ROWS, _C), jnp.float32),
    )(y, rel_max, xt, wg_bd, b_g.reshape(1, 2 * _C), g_bng.reshape(1, 2 * _C),
      b_bng.reshape(1, 2 * _C), W_fc2, b_fc2.reshape(1, _C),
      g_bn2.reshape(1, _C), b_bn2.reshape(1, _C))

    return out.reshape(_B, _N, _C).transpose(0, 2, 1).reshape(_B, _C, _H, _W)


# speedup vs baseline: 1.5205x; 1.1223x over previous
import functools

import numpy as np
import jax
import jax.numpy as jnp
from jax import lax
from jax.experimental import pallas as pl
from jax.experimental.pallas import tpu as pltpu
from jax.experimental.pallas import tpu_sc as plsc

_B, _C, _H, _W = 8, 96, 32, 32
_N = _H * _W
_K = 9
_BN_ROWS = _B * _N
_EPS = 1e-5
_NW = 32
_HB = _B // 2
_RPW = _HB * _N // _NW
_NBUF = 4


def _sincos_1d_np(embed_dim, pos):
    omega = np.arange(embed_dim // 2, dtype=np.float64) / (embed_dim / 2.0)
    omega = 1.0 / (10000.0 ** omega)
    out = pos.reshape(-1)[:, None] * omega[None, :]
    return np.concatenate([np.sin(out), np.cos(out)], axis=1)


def _rel_pos_const(embed_dim, gh, gw):
    gx = np.arange(gw, dtype=np.float64)
    gy = np.arange(gh, dtype=np.float64)
    gX, gY = np.meshgrid(gx, gy)
    emb_w = _sincos_1d_np(embed_dim // 2, gX)
    emb_h = _sincos_1d_np(embed_dim // 2, gY)
    pos = np.concatenate([emb_h, emb_w], axis=1)
    rel = 2.0 * (pos @ pos.T) / float(embed_dim)
    return (-rel).astype(np.float32)


_REL_POS_NP = _rel_pos_const(_C, _H, _W)
_SCALE = 131072.0
_REL_POS_S_NP = (_REL_POS_NP.astype(np.float64) * _SCALE).astype(np.float32)


def _bn_cols(v, gamma, beta):
    mu = jnp.mean(v, axis=0, keepdims=True)
    var = jnp.mean((v - mu) ** 2, axis=0, keepdims=True)
    return (v - mu) / jnp.sqrt(var + _EPS) * gamma + beta


def _dot_bf16(a, b, dims):
    return lax.dot_general(a.astype(jnp.bfloat16), b.astype(jnp.bfloat16),
                           dims, preferred_element_type=jnp.float32)


def _stage_a(xt_ref, w1_ref, b1_ref, g1_ref, bb1_ref, y_ref, yn_ref):
    xt = xt_ref[...]
    y_pre = _dot_bf16(xt, w1_ref[...], (((1,), (1,)), ((), ())))
    y_pre = y_pre + b1_ref[...]
    y = _bn_cols(y_pre, g1_ref[...], bb1_ref[...])
    nrm = jnp.sqrt(jnp.sum(y * y, axis=1, keepdims=True))
    yn = y / jnp.maximum(nrm, 1e-12)
    y_ref[...] = jnp.concatenate([y, jnp.zeros((_BN_ROWS, 128 - _C), jnp.float32)], axis=1)
    yn_ref[...] = yn


def _stage_b(boff, yn_ref, rp_ref, idx_ref):
    yn = yn_ref[...]
    yn2 = yn * yn
    sq_col = jnp.sum(yn2, axis=1, keepdims=True)
    ones_row = jnp.ones((1, _C), jnp.float32)
    s_hi = yn2.astype(jnp.bfloat16)
    r1 = yn2 - s_hi.astype(jnp.float32)
    s_mid = r1.astype(jnp.bfloat16)
    s_lo = r1 - s_mid.astype(jnp.float32)
    dims_t = (((1,), (1,)), ((), ()))
    sq_row = (_dot_bf16(ones_row, s_hi, dims_t)
              + _dot_bf16(ones_row, s_mid, dims_t)
              + _dot_bf16(ones_row, s_lo, dims_t))
    g = _dot_bf16(yn, yn, dims_t)
    ds_ = (sq_col * _SCALE - (2.0 * _SCALE) * g) + (sq_row * _SCALE + rp_ref[...])
    q = ds_.astype(jnp.int32)
    iota = lax.broadcasted_iota(jnp.int32, (_N, _N), 1)
    p = (q << 10) | iota
    lane16 = lax.broadcasted_iota(jnp.int32, (_N, 16), 1)
    base = (pl.program_id(0) + boff) * _N
    acc_idx = jnp.zeros((_N, 16), jnp.int32)
    for k in range(_K):
        m = jnp.min(p, axis=1, keepdims=True)
        idx = m & jnp.int32(1023)
        if k == 0:
            acc_idx = jnp.broadcast_to(idx, (_N, 16))
        else:
            acc_idx = jnp.where(lane16 == k, idx, acc_idx)
        p = jnp.where(p == m, jnp.int32(0x7FFFFFFF), p)
    idx_ref[...] = acc_idx + base


def _sc_gather_max(y_hbm, idx_hbm, out_hbm, idxv, outv,
                   gb0, gb1, gb2, gb3, s0, s1, s2, s3):
    c = lax.axis_index("c")
    s = lax.axis_index("s")
    wid = s * 2 + c
    base = wid * _RPW
    pltpu.sync_copy(idx_hbm.at[pl.ds(base, _RPW)], idxv)
    gbs = (gb0, gb1, gb2, gb3)
    sems = (s0, s1, s2, s3)
    for b in range(_NBUF):
        pltpu.async_copy(y_hbm.at[idxv.at[b, pl.ds(0, _K)]], gbs[b], sems[b])

    def chunk(i, carry):
        g = i * _NBUF
        for b in range(_NBUF):
            r = g + b
            pltpu.make_async_copy(
                y_hbm.at[idxv.at[0, pl.ds(0, _K)]], gbs[b], sems[b]).wait()
            gb = gbs[b]
            for cv in range(_C // 16):
                acc = gb[0, pl.ds(cv * 16, 16)]
                for k in range(1, _K):
                    acc = jnp.maximum(acc, gb[k, pl.ds(cv * 16, 16)])
                outv[r, pl.ds(cv * 16, 16)] = acc

            @pl.when(r + _NBUF < _RPW)
            def _():
                pltpu.async_copy(
                    y_hbm.at[idxv.at[r + _NBUF, pl.ds(0, _K)]], gbs[b], sems[b])
        return carry

    lax.fori_loop(0, _RPW // _NBUF, chunk, 0)
    pltpu.sync_copy(outv, out_hbm.at[pl.ds(base, _RPW)])


def _stage_c(y_ref, rm_ref, xt_ref, wg_ref, bg_ref, gg_ref, bbg_ref,
             w2_ref, b2_ref, g2_ref, bb2_ref, out_ref):
    y = y_ref[...][:, :_C]
    h = jnp.concatenate([y, rm_ref[...] - y], axis=1)
    hg = _dot_bf16(h, wg_ref[...], (((1,), (0,)), ((), ()))) + bg_ref[...]
    hg = _bn_cols(hg, gg_ref[...], bbg_ref[...])
    hg = 0.5 * hg * (1.0 + lax.erf(hg * np.float32(1.0 / np.sqrt(2.0))))
    out = _dot_bf16(hg, w2_ref[...], (((1,), (1,)), ((), ()))) + b2_ref[...]
    out = _bn_cols(out, g2_ref[...], bb2_ref[...])
    out_ref[...] = out + xt_ref[...]


def kernel(x, W_fc1, b_fc1, g_bn1, b_bn1, W_g, b_g, g_bng, b_bng,
           W_fc2, b_fc2, g_bn2, b_bn2):
    xt = x.reshape(_B, _C, _N).transpose(0, 2, 1).reshape(_BN_ROWS, _C)
    rel_pos_s = jnp.asarray(_REL_POS_S_NP)

    y, yn = pl.pallas_call(
        _stage_a,
        out_shape=[jax.ShapeDtypeStruct((_BN_ROWS, 128), jnp.float32),
                   jax.ShapeDtypeStruct((_BN_ROWS, _C), jnp.float32)],
    )(xt, W_fc1, b_fc1.reshape(1, _C), g_bn1.reshape(1, _C),
      b_bn1.reshape(1, _C))

    mesh = plsc.VectorSubcoreMesh(core_axis_name="c", subcore_axis_name="s")
    sc_scratch = [
        pltpu.VMEM((_RPW, 16), jnp.int32),
        pltpu.VMEM((_RPW, _C), jnp.float32),
        pltpu.VMEM((_K, 128), jnp.float32),
        pltpu.VMEM((_K, 128), jnp.float32),
        pltpu.VMEM((_K, 128), jnp.float32),
        pltpu.VMEM((_K, 128), jnp.float32),
        pltpu.SemaphoreType.DMA,
        pltpu.SemaphoreType.DMA,
        pltpu.SemaphoreType.DMA,
        pltpu.SemaphoreType.DMA,
    ]

    halves = []
    for h in range(2):
        boff = h * _HB
        nn_idx_h = pl.pallas_call(
            functools.partial(_stage_b, boff),
            grid=(_HB,),
            in_specs=[
                pl.BlockSpec((_N, _C), lambda i, boff=boff: (i + boff, 0)),
                pl.BlockSpec((_N, _N), lambda i: (0, 0)),
            ],
            out_specs=pl.BlockSpec((_N, 16), lambda i: (i, 0)),
            out_shape=jax.ShapeDtypeStruct((_HB * _N, 16), jnp.int32),
            compiler_params=pltpu.CompilerParams(
                dimension_semantics=("arbitrary",)),
        )(yn, rel_pos_s)
        rm_h = pl.kernel(
            _sc_gather_max,
            mesh=mesh,
            out_type=jax.ShapeDtypeStruct((_HB * _N, _C), jnp.float32),
            scratch_types=sc_scratch,
        )(y, nn_idx_h)
        halves.append(rm_h)

    rel_max = jnp.concatenate(halves, axis=0)

    wg_bd = jax.scipy.linalg.block_diag(*[W_g[g].T for g in range(4)])

    out = pl.pallas_call(
        _stage_c,
        out_shape=jax.ShapeDtypeStruct((_BN_ROWS, _C), jnp.float32),
    )(y, rel_max, xt, wg_bd, b_g.reshape(1, 2 * _C), g_bng.reshape(1, 2 * _C),
      b_bng.reshape(1, 2 * _C), W_fc2, b_fc2.reshape(1, _C),
      g_bn2.reshape(1, _C), b_bn2.reshape(1, _C))

    return out.reshape(_B, _N, _C).transpose(0, 2, 1).reshape(_B, _C, _H, _W)


# speedup vs baseline: 1.6318x; 1.0733x over previous
import functools

import numpy as np
import jax
import jax.numpy as jnp
from jax import lax
from jax.experimental import pallas as pl
from jax.experimental.pallas import tpu as pltpu
from jax.experimental.pallas import tpu_sc as plsc

_B, _C, _H, _W = 8, 96, 32, 32
_N = _H * _W
_K = 9
_BN_ROWS = _B * _N
_EPS = 1e-5
_NW = 32
_NSPLIT = 4
_HB = _B // _NSPLIT
_RPW = _HB * _N // _NW
_NBUF = 4


def _sincos_1d_np(embed_dim, pos):
    omega = np.arange(embed_dim // 2, dtype=np.float64) / (embed_dim / 2.0)
    omega = 1.0 / (10000.0 ** omega)
    out = pos.reshape(-1)[:, None] * omega[None, :]
    return np.concatenate([np.sin(out), np.cos(out)], axis=1)


def _rel_pos_const(embed_dim, gh, gw):
    gx = np.arange(gw, dtype=np.float64)
    gy = np.arange(gh, dtype=np.float64)
    gX, gY = np.meshgrid(gx, gy)
    emb_w = _sincos_1d_np(embed_dim // 2, gX)
    emb_h = _sincos_1d_np(embed_dim // 2, gY)
    pos = np.concatenate([emb_h, emb_w], axis=1)
    rel = 2.0 * (pos @ pos.T) / float(embed_dim)
    return (-rel).astype(np.float32)


_REL_POS_NP = _rel_pos_const(_C, _H, _W)
_SCALE = 131072.0
_REL_POS_S_NP = (_REL_POS_NP.astype(np.float64) * _SCALE).astype(np.float32)


def _bn_cols(v, gamma, beta):
    mu = jnp.mean(v, axis=0, keepdims=True)
    var = jnp.mean(v * v, axis=0, keepdims=True) - mu * mu
    return (v - mu) / jnp.sqrt(var + _EPS) * gamma + beta


def _dot_bf16(a, b, dims):
    return lax.dot_general(a.astype(jnp.bfloat16), b.astype(jnp.bfloat16),
                           dims, preferred_element_type=jnp.float32)


def _stage_a(xt_ref, w1_ref, b1_ref, g1_ref, bb1_ref, y_ref, yn_ref):
    xt = xt_ref[...]
    y_pre = _dot_bf16(xt, w1_ref[...], (((1,), (1,)), ((), ())))
    y_pre = y_pre + b1_ref[...]
    y = _bn_cols(y_pre, g1_ref[...], bb1_ref[...])
    nrm = jnp.sqrt(jnp.sum(y * y, axis=1, keepdims=True))
    yn = y / jnp.maximum(nrm, 1e-12)
    y_ref[...] = jnp.concatenate([y, jnp.zeros((_BN_ROWS, 128 - _C), jnp.float32)], axis=1)
    yn_ref[...] = yn


def _stage_b(boff, yn_ref, rp_ref, idx_ref):
    yn = yn_ref[...]
    yn2 = yn * yn
    sq_col = jnp.sum(yn2, axis=1, keepdims=True)
    ones_row = jnp.ones((1, _C), jnp.float32)
    s_hi = yn2.astype(jnp.bfloat16)
    r1 = yn2 - s_hi.astype(jnp.float32)
    s_mid = r1.astype(jnp.bfloat16)
    s_lo = r1 - s_mid.astype(jnp.float32)
    dims_t = (((1,), (1,)), ((), ()))
    sq_row = (_dot_bf16(ones_row, s_hi, dims_t)
              + _dot_bf16(ones_row, s_mid, dims_t)
              + _dot_bf16(ones_row, s_lo, dims_t))
    g = _dot_bf16(yn, yn, dims_t)
    ds_ = (sq_col * _SCALE - (2.0 * _SCALE) * g) + (sq_row * _SCALE + rp_ref[...])
    q = ds_.astype(jnp.int32)
    iota = lax.broadcasted_iota(jnp.int32, (_N, _N), 1)
    p = (q << 10) | iota
    lane16 = lax.broadcasted_iota(jnp.int32, (_N, 16), 1)
    base = (pl.program_id(0) + boff) * _N
    acc_idx = jnp.zeros((_N, 16), jnp.int32)
    for k in range(_K):
        m = jnp.min(p, axis=1, keepdims=True)
        idx = m & jnp.int32(1023)
        if k == 0:
            acc_idx = jnp.broadcast_to(idx, (_N, 16))
        else:
            acc_idx = jnp.where(lane16 == k, idx, acc_idx)
        p = jnp.where(p == m, jnp.int32(0x7FFFFFFF), p)
    idx_ref[...] = acc_idx + base


def _sc_gather_max(y_hbm, idx_hbm, out_hbm, idxv, outv,
                   gb0, gb1, gb2, gb3, s0, s1, s2, s3):
    c = lax.axis_index("c")
    s = lax.axis_index("s")
    wid = s * 2 + c
    base = wid * _RPW
    pltpu.sync_copy(idx_hbm.at[pl.ds(base, _RPW)], idxv)
    gbs = (gb0, gb1, gb2, gb3)
    sems = (s0, s1, s2, s3)
    for b in range(_NBUF):
        pltpu.async_copy(y_hbm.at[idxv.at[b, pl.ds(0, _K)]], gbs[b], sems[b])

    def chunk(i, carry):
        g = i * _NBUF
        for b in range(_NBUF):
            r = g + b
            pltpu.make_async_copy(
                y_hbm.at[idxv.at[0, pl.ds(0, _K)]], gbs[b], sems[b]).wait()
            gb = gbs[b]
            for cv in range(_C // 16):
                acc = gb[0, pl.ds(cv * 16, 16)]
                for k in range(1, _K):
                    acc = jnp.maximum(acc, gb[k, pl.ds(cv * 16, 16)])
                outv[r, pl.ds(cv * 16, 16)] = acc

            @pl.when(r + _NBUF < _RPW)
            def _():
                pltpu.async_copy(
                    y_hbm.at[idxv.at[r + _NBUF, pl.ds(0, _K)]], gbs[b], sems[b])
        return carry

    lax.fori_loop(0, _RPW // _NBUF, chunk, 0)
    pltpu.sync_copy(outv, out_hbm.at[pl.ds(base, _RPW)])


def _stage_c(y_ref, rm0_ref, rm1_ref, rm2_ref, rm3_ref, xt_ref, wg_ref,
             bg_ref, gg_ref, bbg_ref, w2_ref, b2_ref, g2_ref, bb2_ref,
             out_ref):
    y = y_ref[...][:, :_C]
    rm = jnp.concatenate([rm0_ref[...], rm1_ref[...],
                          rm2_ref[...], rm3_ref[...]], axis=0)
    h = jnp.concatenate([y, rm - y], axis=1)
    hg = _dot_bf16(h, wg_ref[...], (((1,), (0,)), ((), ()))) + bg_ref[...]
    hg = _bn_cols(hg, gg_ref[...], bbg_ref[...])
    hg = 0.5 * hg * (1.0 + lax.erf(hg * np.float32(1.0 / np.sqrt(2.0))))
    out = _dot_bf16(hg, w2_ref[...], (((1,), (1,)), ((), ()))) + b2_ref[...]
    out = _bn_cols(out, g2_ref[...], bb2_ref[...])
    out_ref[...] = out + xt_ref[...]


def kernel(x, W_fc1, b_fc1, g_bn1, b_bn1, W_g, b_g, g_bng, b_bng,
           W_fc2, b_fc2, g_bn2, b_bn2):
    xt = x.reshape(_B, _C, _N).transpose(0, 2, 1).reshape(_BN_ROWS, _C)
    rel_pos_s = jnp.asarray(_REL_POS_S_NP)

    y, yn = pl.pallas_call(
        _stage_a,
        out_shape=[jax.ShapeDtypeStruct((_BN_ROWS, 128), jnp.float32),
                   jax.ShapeDtypeStruct((_BN_ROWS, _C), jnp.float32)],
    )(xt, W_fc1, b_fc1.reshape(1, _C), g_bn1.reshape(1, _C),
      b_bn1.reshape(1, _C))

    mesh = plsc.VectorSubcoreMesh(core_axis_name="c", subcore_axis_name="s")
    sc_scratch = [
        pltpu.VMEM((_RPW, 16), jnp.int32),
        pltpu.VMEM((_RPW, _C), jnp.float32),
        pltpu.VMEM((_K, 128), jnp.float32),
        pltpu.VMEM((_K, 128), jnp.float32),
        pltpu.VMEM((_K, 128), jnp.float32),
        pltpu.VMEM((_K, 128), jnp.float32),
        pltpu.SemaphoreType.DMA,
        pltpu.SemaphoreType.DMA,
        pltpu.SemaphoreType.DMA,
        pltpu.SemaphoreType.DMA,
    ]

    halves = []
    for h in range(_NSPLIT):
        boff = h * _HB
        nn_idx_h = pl.pallas_call(
            functools.partial(_stage_b, boff),
            grid=(_HB,),
            in_specs=[
                pl.BlockSpec((_N, _C), lambda i, boff=boff: (i + boff, 0)),
                pl.BlockSpec((_N, _N), lambda i: (0, 0)),
            ],
            out_specs=pl.BlockSpec((_N, 16), lambda i: (i, 0)),
            out_shape=jax.ShapeDtypeStruct((_HB * _N, 16), jnp.int32),
            compiler_params=pltpu.CompilerParams(
                dimension_semantics=("arbitrary",)),
        )(yn, rel_pos_s)
        rm_h = pl.kernel(
            _sc_gather_max,
            mesh=mesh,
            out_type=jax.ShapeDtypeStruct((_HB * _N, _C), jnp.float32),
            scratch_types=sc_scratch,
        )(y, nn_idx_h)
        halves.append(rm_h)

    wg_bd = jax.scipy.linalg.block_diag(*[W_g[g].T for g in range(4)])

    out = pl.pallas_call(
        _stage_c,
        out_shape=jax.ShapeDtypeStruct((_BN_ROWS, _C), jnp.float32),
    )(y, *halves, xt, wg_bd, b_g.reshape(1, 2 * _C), g_bng.reshape(1, 2 * _C),
      b_bng.reshape(1, 2 * _C), W_fc2, b_fc2.reshape(1, _C),
      g_bn2.reshape(1, _C), b_bn2.reshape(1, _C))

    return out.reshape(_B, _N, _C).transpose(0, 2, 1).reshape(_B, _C, _H, _W)
